# Initial kernel scaffold; baseline (speedup 1.0000x reference)
import jax
import jax.numpy as jnp
from jax.experimental import pallas as pl


def kernel(x, edge_index, W, b):
    raise NotImplementedError("write your pallas kernel here")


# speedup vs baseline: 7.1840x; 7.1840x over previous
import functools

import jax
import jax.numpy as jnp
from jax import lax
from jax.experimental import pallas as pl
from jax.experimental.pallas import tpu as pltpu
from jax.experimental.pallas import tpu_sc as plsc

N_NODES = 10000
N_EDGES = 320000
FEATS = 128

NC = 2
NS = 16
NW = NC * NS

CHUNK = 128
CHUNKS_PER_W = -(-N_EDGES // (NW * CHUNK))
EDGES_PER_W = CHUNKS_PER_W * CHUNK
E_PAD = EDGES_PER_W * NW

NP = 10240
ROWS_PER_TILE = NP // NS
DEG_W = 16

_mesh = plsc.VectorSubcoreMesh(core_axis_name="c", subcore_axis_name="s")


@functools.partial(
    pl.kernel,
    out_type=[
        jax.ShapeDtypeStruct((NC, NP, FEATS), jnp.float32),
        jax.ShapeDtypeStruct((NC * NP,), jnp.float32),
    ],
    mesh=_mesh,
    scratch_types=[
        pltpu.VMEM((CHUNK,), jnp.int32),
        pltpu.VMEM((CHUNK,), jnp.int32),
        pltpu.VMEM((CHUNK, FEATS), jnp.float32),
        pltpu.VMEM((CHUNK,), jnp.float32),
        pltpu.VMEM((ROWS_PER_TILE,), jnp.float32),
        pltpu.SemaphoreType.DMA,
        pltpu.VMEM_SHARED((NP, FEATS), jnp.float32),
        pltpu.VMEM_SHARED((NP,), jnp.float32),
    ],
)
def _sc_aggregate(x_hbm, row_hbm, col_hbm, zf_hbm,
                  sum_out, deg_out,
                  colv, rowv, rows_v, onesv, degv, sem, accum_sh, deg_sh):
    c = lax.axis_index("c")
    s = lax.axis_index("s")
    wid = s * NC + c

    @pl.when(s == 0)
    def _zero():
        pltpu.sync_copy(zf_hbm, accum_sh)

    def _fill_ones(i, carry):
        onesv[pl.ds(i * 16, 16)] = jnp.ones((16,), jnp.float32)
        return carry
    lax.fori_loop(0, CHUNK // 16, _fill_ones, 0)

    def _zero_deg(i, carry):
        degv[pl.ds(i * 16, 16)] = jnp.zeros((16,), jnp.float32)
        return carry
    lax.fori_loop(0, ROWS_PER_TILE // 16, _zero_deg, 0)
    r0 = s * ROWS_PER_TILE
    pltpu.sync_copy(degv, deg_sh.at[pl.ds(r0, ROWS_PER_TILE)])

    plsc.subcore_barrier()

    base0 = wid * EDGES_PER_W

    def _edge_chunk(k, carry):
        off = base0 + k * CHUNK
        pltpu.sync_copy(col_hbm.at[pl.ds(off, CHUNK)], colv)
        pltpu.async_copy(x_hbm.at[colv], rows_v, sem).wait()
        pltpu.sync_copy(row_hbm.at[pl.ds(off, CHUNK)], rowv)
        pltpu.sync_copy(rows_v, accum_sh.at[rowv], add=True)
        pltpu.sync_copy(onesv, deg_sh.at[rowv], add=True)
        return carry
    lax.fori_loop(0, CHUNKS_PER_W, _edge_chunk, 0)

    plsc.subcore_barrier()

    pltpu.sync_copy(accum_sh.at[pl.ds(r0, ROWS_PER_TILE)],
                    sum_out.at[c, pl.ds(r0, ROWS_PER_TILE)])
    pltpu.sync_copy(deg_sh.at[pl.ds(r0, ROWS_PER_TILE)], degv)
    pltpu.sync_copy(degv, deg_out.at[pl.ds(c * NP + r0, ROWS_PER_TILE)])


ROW_BLK = 400
N_BLKS = N_NODES // ROW_BLK


def _tc_project(sp_ref, dp_ref, wt_ref, b_ref, o_ref):
    s = sp_ref[0] + sp_ref[1]
    d = dp_ref[0] + dp_ref[1]
    agg = s / jnp.maximum(d, 1.0)
    o_ref[...] = (
        jnp.dot(agg, wt_ref[...], preferred_element_type=jnp.float32)
        + b_ref[...]
    )


def kernel(x, edge_index, W, b):
    row = edge_index[0].astype(jnp.int32)
    col = edge_index[1].astype(jnp.int32)
    pad = E_PAD - N_EDGES
    pad_iota = jnp.arange(pad, dtype=jnp.int32)
    rowp = jnp.concatenate([row, N_NODES + (pad_iota % (NP - N_NODES))])
    colp = jnp.concatenate([col, pad_iota % N_NODES])
    zf = jnp.zeros((NP, FEATS), jnp.float32)

    sum_p, deg_p = _sc_aggregate(x, rowp, colp, zf)

    out = pl.pallas_call(
        _tc_project,
        grid=(N_BLKS,),
        in_specs=[
            pl.BlockSpec((NC, ROW_BLK, FEATS), lambda i: (0, i, 0)),
            pl.BlockSpec((NC, ROW_BLK, 1), lambda i: (0, i, 0)),
            pl.BlockSpec((FEATS, FEATS), lambda i: (0, 0)),
            pl.BlockSpec((1, FEATS), lambda i: (0, 0)),
        ],
        out_specs=pl.BlockSpec((ROW_BLK, FEATS), lambda i: (i, 0)),
        out_shape=jax.ShapeDtypeStruct((N_NODES, FEATS), jnp.float32),
    )(sum_p, deg_p.reshape(NC, NP, 1), W.T, b.reshape(1, FEATS))
    return out


# speedup vs baseline: 11.4015x; 1.5871x over previous
import functools

import jax
import jax.numpy as jnp
from jax import lax
from jax.experimental import pallas as pl
from jax.experimental.pallas import tpu as pltpu
from jax.experimental.pallas import tpu_sc as plsc

N_NODES = 10000
N_EDGES = 320000
FEATS = 128

NC = 2
NS = 16
NW = NC * NS

CHUNK = 128
CPW = 80
EDGES_PER_W = CPW * CHUNK
E_PAD = EDGES_PER_W * NW

NP = 10240
ROWS_PER_TILE = NP // NS

_mesh = plsc.VectorSubcoreMesh(core_axis_name="c", subcore_axis_name="s")


@functools.partial(
    pl.kernel,
    out_type=[
        jax.ShapeDtypeStruct((NC, NP, FEATS), jnp.float32),
        jax.ShapeDtypeStruct((NC * NP,), jnp.float32),
    ],
    mesh=_mesh,
    scratch_types=[
        pltpu.VMEM((CPW // 2, CHUNK), jnp.int32),
        pltpu.VMEM((CPW // 2, CHUNK), jnp.int32),
        pltpu.VMEM((CHUNK, FEATS), jnp.float32),
        pltpu.VMEM((CHUNK, FEATS), jnp.float32),
        pltpu.VMEM((CHUNK,), jnp.float32),
        pltpu.VMEM((ROWS_PER_TILE,), jnp.float32),
        pltpu.SemaphoreType.DMA,
        pltpu.SemaphoreType.DMA,
        pltpu.VMEM_SHARED((NP, FEATS), jnp.float32),
        pltpu.VMEM_SHARED((NP,), jnp.float32),
    ],
)
def _sc_aggregate(x_hbm, row_hbm, col_hbm, zf_hbm,
                  sum_out, deg_out,
                  idxc, idxr, buf0, buf1, onesv, degv, sem0, sem1,
                  accum_sh, deg_sh):
    c = lax.axis_index("c")
    s = lax.axis_index("s")
    wid = s * NC + c
    r0 = s * ROWS_PER_TILE

    pltpu.sync_copy(zf_hbm.at[pl.ds(r0, ROWS_PER_TILE)],
                    accum_sh.at[pl.ds(r0, ROWS_PER_TILE)])

    def _zero_deg(i, carry):
        degv[pl.ds(i * 16, 16)] = jnp.zeros((16,), jnp.float32)
        return carry
    lax.fori_loop(0, ROWS_PER_TILE // 16, _zero_deg, 0)
    pltpu.sync_copy(degv, deg_sh.at[pl.ds(r0, ROWS_PER_TILE)])

    def _fill_ones(t, carry):
        onesv[pl.ds(t * 16, 16)] = jnp.ones((16,), jnp.float32)
        return carry
    lax.fori_loop(0, CHUNK // 16, _fill_ones, 0)

    plsc.subcore_barrier()

    HC = CPW // 2
    for h in range(2):
        pltpu.sync_copy(col_hbm.at[wid, pl.ds(h * HC, HC)], idxc)
        pltpu.sync_copy(row_hbm.at[wid, pl.ds(h * HC, HC)], idxr)
        pltpu.async_copy(x_hbm.at[idxc.at[0]], buf0, sem0)

        def _edge_pair(g, carry):
            k0 = 2 * g
            k1 = 2 * g + 1
            k2 = jnp.minimum(2 * g + 2, HC - 1)
            pltpu.make_async_copy(x_hbm.at[idxc.at[k0]], buf0, sem0).wait()
            pltpu.async_copy(x_hbm.at[idxc.at[k1]], buf1, sem1)
            pltpu.sync_copy(buf0, accum_sh.at[idxr.at[k0]], add=True)
            pltpu.sync_copy(onesv, deg_sh.at[idxr.at[k0]], add=True)
            pltpu.make_async_copy(x_hbm.at[idxc.at[k1]], buf1, sem1).wait()
            pltpu.async_copy(x_hbm.at[idxc.at[k2]], buf0, sem0)
            pltpu.sync_copy(buf1, accum_sh.at[idxr.at[k1]], add=True)
            pltpu.sync_copy(onesv, deg_sh.at[idxr.at[k1]], add=True)
            return carry
        lax.fori_loop(0, HC // 2, _edge_pair, 0)

        pltpu.make_async_copy(x_hbm.at[idxc.at[HC - 1]], buf0, sem0).wait()

    plsc.subcore_barrier()

    pltpu.sync_copy(accum_sh.at[pl.ds(r0, ROWS_PER_TILE)],
                    sum_out.at[c, pl.ds(r0, ROWS_PER_TILE)])
    pltpu.sync_copy(deg_sh.at[pl.ds(r0, ROWS_PER_TILE)], degv)
    pltpu.sync_copy(degv, deg_out.at[pl.ds(c * NP + r0, ROWS_PER_TILE)])


ROW_BLK = 400
N_BLKS = N_NODES // ROW_BLK


def _tc_project(sp_ref, dp_ref, wt_ref, b_ref, o_ref):
    s = sp_ref[0] + sp_ref[1]
    d = dp_ref[0] + dp_ref[1]
    agg = s / jnp.maximum(d, 1.0)
    o_ref[...] = (
        jnp.dot(agg, wt_ref[...], preferred_element_type=jnp.float32)
        + b_ref[...]
    )


def kernel(x, edge_index, W, b):
    row = edge_index[0].astype(jnp.int32)
    col = edge_index[1].astype(jnp.int32)
    pad = E_PAD - N_EDGES
    pad_iota = jnp.arange(pad, dtype=jnp.int32)
    rowp = jnp.concatenate([row, N_NODES + (pad_iota % (NP - N_NODES))])
    colp = jnp.concatenate([col, pad_iota % N_NODES])
    rowp = rowp.reshape(NW, CPW, CHUNK)
    colp = colp.reshape(NW, CPW, CHUNK)
    zf = jnp.zeros((NP, FEATS), jnp.float32)

    sum_p, deg_p = _sc_aggregate(x, rowp, colp, zf)

    out = pl.pallas_call(
        _tc_project,
        grid=(N_BLKS,),
        in_specs=[
            pl.BlockSpec((NC, ROW_BLK, FEATS), lambda i: (0, i, 0)),
            pl.BlockSpec((NC, ROW_BLK, 1), lambda i: (0, i, 0)),
            pl.BlockSpec((FEATS, FEATS), lambda i: (0, 0)),
            pl.BlockSpec((1, FEATS), lambda i: (0, 0)),
        ],
        out_specs=pl.BlockSpec((ROW_BLK, FEATS), lambda i: (i, 0)),
        out_shape=jax.ShapeDtypeStruct((N_NODES, FEATS), jnp.float32),
    )(sum_p, deg_p.reshape(NC, NP, 1), W.T, b.reshape(1, FEATS))
    return out


# speedup vs baseline: 11.6864x; 1.0250x over previous
import functools

import jax
import jax.numpy as jnp
from jax import lax
from jax.experimental import pallas as pl
from jax.experimental.pallas import tpu as pltpu
from jax.experimental.pallas import tpu_sc as plsc

N_NODES = 10000
N_EDGES = 320000
FEATS = 128

NC = 2
NS = 16
NW = NC * NS

CHUNK = 128
CPW = 80
EDGES_PER_W = CPW * CHUNK
E_PAD = EDGES_PER_W * NW

NP = 10240
ROWS_PER_TILE = NP // NS

_mesh = plsc.VectorSubcoreMesh(core_axis_name="c", subcore_axis_name="s")


@functools.partial(
    pl.kernel,
    out_type=[
        jax.ShapeDtypeStruct((NC, NP, FEATS), jnp.float32),
        jax.ShapeDtypeStruct((NC * NP,), jnp.float32),
    ],
    mesh=_mesh,
    scratch_types=[
        pltpu.VMEM((CPW // 2, CHUNK), jnp.int32),
        pltpu.VMEM((CPW // 2, CHUNK), jnp.int32),
        pltpu.VMEM((CHUNK, FEATS), jnp.float32),
        pltpu.VMEM((CHUNK, FEATS), jnp.float32),
        pltpu.VMEM((CHUNK,), jnp.float32),
        pltpu.VMEM((ROWS_PER_TILE,), jnp.float32),
        pltpu.SemaphoreType.DMA,
        pltpu.SemaphoreType.DMA,
        pltpu.SemaphoreType.DMA,
        pltpu.VMEM_SHARED((NP, FEATS), jnp.float32),
        pltpu.VMEM_SHARED((NP,), jnp.float32),
    ],
)
def _sc_aggregate(x_hbm, row_hbm, col_hbm,
                  sum_out, deg_out,
                  idxc, idxr, buf0, buf1, onesv, degv, sem0, sem1, sem2,
                  accum_sh, deg_sh):
    c = lax.axis_index("c")
    s = lax.axis_index("s")
    wid = s * NC + c
    r0 = s * ROWS_PER_TILE

    def _zero_buf(i, carry):
        for j in range(8):
            buf0[i, pl.ds(j * 16, 16)] = jnp.zeros((16,), jnp.float32)
        return carry
    lax.fori_loop(0, CHUNK, _zero_buf, 0)
    for j in range(ROWS_PER_TILE // CHUNK):
        pltpu.sync_copy(buf0, accum_sh.at[pl.ds(r0 + j * CHUNK, CHUNK)])

    def _zero_deg(i, carry):
        degv[pl.ds(i * 16, 16)] = jnp.zeros((16,), jnp.float32)
        return carry
    lax.fori_loop(0, ROWS_PER_TILE // 16, _zero_deg, 0)
    pltpu.sync_copy(degv, deg_sh.at[pl.ds(r0, ROWS_PER_TILE)])

    def _fill_ones(t, carry):
        onesv[pl.ds(t * 16, 16)] = jnp.ones((16,), jnp.float32)
        return carry
    lax.fori_loop(0, CHUNK // 16, _fill_ones, 0)

    plsc.subcore_barrier()

    HC = CPW // 2
    for h in range(2):
        pltpu.sync_copy(col_hbm.at[wid, pl.ds(h * HC, HC)], idxc)
        pltpu.sync_copy(row_hbm.at[wid, pl.ds(h * HC, HC)], idxr)
        pltpu.async_copy(x_hbm.at[idxc.at[0]], buf0, sem0)

        def _edge_pair(g, carry):
            k0 = 2 * g
            k1 = 2 * g + 1
            k2 = jnp.minimum(2 * g + 2, HC - 1)
            pltpu.make_async_copy(x_hbm.at[idxc.at[k0]], buf0, sem0).wait()
            pltpu.async_copy(x_hbm.at[idxc.at[k1]], buf1, sem1)
            pltpu.sync_copy(buf0, accum_sh.at[idxr.at[k0]], add=True)
            pltpu.async_copy(onesv, deg_sh.at[idxr.at[k0]], sem2, add=True)
            pltpu.make_async_copy(x_hbm.at[idxc.at[k1]], buf1, sem1).wait()
            pltpu.async_copy(x_hbm.at[idxc.at[k2]], buf0, sem0)
            pltpu.sync_copy(buf1, accum_sh.at[idxr.at[k1]], add=True)
            pltpu.async_copy(onesv, deg_sh.at[idxr.at[k1]], sem2, add=True)
            return carry
        lax.fori_loop(0, HC // 2, _edge_pair, 0)

        pltpu.make_async_copy(x_hbm.at[idxc.at[HC - 1]], buf0, sem0).wait()

        def _drain_deg(k, carry):
            pltpu.make_async_copy(onesv, deg_sh.at[idxr.at[k]], sem2).wait()
            return carry
        lax.fori_loop(0, HC, _drain_deg, 0)

    plsc.subcore_barrier()

    pltpu.sync_copy(accum_sh.at[pl.ds(r0, ROWS_PER_TILE)],
                    sum_out.at[c, pl.ds(r0, ROWS_PER_TILE)])
    pltpu.sync_copy(deg_sh.at[pl.ds(r0, ROWS_PER_TILE)], degv)
    pltpu.sync_copy(degv, deg_out.at[pl.ds(c * NP + r0, ROWS_PER_TILE)])


ROW_BLK = 400
N_BLKS = N_NODES // ROW_BLK


def _tc_project(sp_ref, dp_ref, wt_ref, b_ref, o_ref):
    s = sp_ref[0] + sp_ref[1]
    d = dp_ref[0] + dp_ref[1]
    agg = s / jnp.maximum(d, 1.0)
    o_ref[...] = (
        jnp.dot(agg, wt_ref[...], preferred_element_type=jnp.float32)
        + b_ref[...]
    )


def kernel(x, edge_index, W, b):
    row = edge_index[0].astype(jnp.int32)
    col = edge_index[1].astype(jnp.int32)
    pad = E_PAD - N_EDGES
    pad_iota = jnp.arange(pad, dtype=jnp.int32)
    rowp = jnp.concatenate([row, N_NODES + (pad_iota % (NP - N_NODES))])
    colp = jnp.concatenate([col, pad_iota % N_NODES])
    rowp = rowp.reshape(NW, CPW, CHUNK)
    colp = colp.reshape(NW, CPW, CHUNK)

    sum_p, deg_p = _sc_aggregate(x, rowp, colp)

    out = pl.pallas_call(
        _tc_project,
        grid=(N_BLKS,),
        in_specs=[
            pl.BlockSpec((NC, ROW_BLK, FEATS), lambda i: (0, i, 0)),
            pl.BlockSpec((NC, ROW_BLK, 1), lambda i: (0, i, 0)),
            pl.BlockSpec((FEATS, FEATS), lambda i: (0, 0)),
            pl.BlockSpec((1, FEATS), lambda i: (0, 0)),
        ],
        out_specs=pl.BlockSpec((ROW_BLK, FEATS), lambda i: (i, 0)),
        out_shape=jax.ShapeDtypeStruct((N_NODES, FEATS), jnp.float32),
    )(sum_p, deg_p.reshape(NC, NP, 1), W.T, b.reshape(1, FEATS))
    return out


# speedup vs baseline: 12.7643x; 1.0922x over previous
import functools

import jax
import jax.numpy as jnp
from jax import lax
from jax.experimental import pallas as pl
from jax.experimental.pallas import tpu as pltpu
from jax.experimental.pallas import tpu_sc as plsc

N_NODES = 10000
N_EDGES = 320000
FEATS = 128

NC = 2
NS = 16
NW = NC * NS

CHUNK = 128
CPW = 80
EDGES_PER_W = CPW * CHUNK
E_PAD = EDGES_PER_W * NW

NP = 10240
ROWS_PER_TILE = NP // NS

_mesh = plsc.VectorSubcoreMesh(core_axis_name="c", subcore_axis_name="s")


@functools.partial(
    pl.kernel,
    out_type=[
        jax.ShapeDtypeStruct((NC, NP, FEATS), jnp.float32),
        jax.ShapeDtypeStruct((NC * NP,), jnp.float32),
    ],
    mesh=_mesh,
    scratch_types=[
        pltpu.VMEM((CPW // 2, CHUNK), jnp.int32),
        pltpu.VMEM((CPW // 2, CHUNK), jnp.int32),
        pltpu.VMEM((CHUNK, FEATS), jnp.float32),
        pltpu.VMEM((CHUNK, FEATS), jnp.float32),
        pltpu.VMEM((CHUNK,), jnp.float32),
        pltpu.VMEM((ROWS_PER_TILE,), jnp.float32),
        pltpu.SemaphoreType.DMA,
        pltpu.SemaphoreType.DMA,
        pltpu.SemaphoreType.DMA,
        pltpu.VMEM_SHARED((NP, FEATS), jnp.float32),
        pltpu.VMEM_SHARED((NP,), jnp.float32),
    ],
)
def _sc_aggregate(x_hbm, edges_hbm,
                  sum_out, deg_out,
                  idxc, idxr, buf0, buf1, onesv, degv, sem0, sem1, sem2,
                  accum_sh, deg_sh):
    c = lax.axis_index("c")
    s = lax.axis_index("s")
    wid = s * NC + c
    r0 = s * ROWS_PER_TILE

    def _zero_buf(i, carry):
        for j in range(8):
            buf0[i, pl.ds(j * 16, 16)] = jnp.zeros((16,), jnp.float32)
        return carry
    lax.fori_loop(0, CHUNK, _zero_buf, 0)
    for j in range(ROWS_PER_TILE // CHUNK):
        pltpu.sync_copy(buf0, accum_sh.at[pl.ds(r0 + j * CHUNK, CHUNK)])

    def _zero_deg(i, carry):
        degv[pl.ds(i * 16, 16)] = jnp.zeros((16,), jnp.float32)
        return carry
    lax.fori_loop(0, ROWS_PER_TILE // 16, _zero_deg, 0)
    pltpu.sync_copy(degv, deg_sh.at[pl.ds(r0, ROWS_PER_TILE)])

    def _fill_ones(t, carry):
        onesv[pl.ds(t * 16, 16)] = jnp.ones((16,), jnp.float32)
        return carry
    lax.fori_loop(0, CHUNK // 16, _fill_ones, 0)

    plsc.subcore_barrier()

    HC = CPW // 2
    for h in range(2):
        pltpu.sync_copy(edges_hbm.at[1, wid, pl.ds(h * HC, HC)], idxc)
        pltpu.sync_copy(edges_hbm.at[0, wid, pl.ds(h * HC, HC)], idxr)
        pltpu.async_copy(x_hbm.at[idxc.at[0]], buf0, sem0)

        def _edge_pair(g, carry):
            k0 = 2 * g
            k1 = 2 * g + 1
            k2 = jnp.minimum(2 * g + 2, HC - 1)
            pltpu.make_async_copy(x_hbm.at[idxc.at[k0]], buf0, sem0).wait()
            pltpu.async_copy(x_hbm.at[idxc.at[k1]], buf1, sem1)
            pltpu.sync_copy(buf0, accum_sh.at[idxr.at[k0]], add=True)
            pltpu.async_copy(onesv, deg_sh.at[idxr.at[k0]], sem2, add=True)
            pltpu.make_async_copy(x_hbm.at[idxc.at[k1]], buf1, sem1).wait()
            pltpu.async_copy(x_hbm.at[idxc.at[k2]], buf0, sem0)
            pltpu.sync_copy(buf1, accum_sh.at[idxr.at[k1]], add=True)
            pltpu.async_copy(onesv, deg_sh.at[idxr.at[k1]], sem2, add=True)
            return carry
        lax.fori_loop(0, HC // 2, _edge_pair, 0)

        pltpu.make_async_copy(x_hbm.at[idxc.at[HC - 1]], buf0, sem0).wait()

        def _drain_deg(k, carry):
            pltpu.make_async_copy(onesv, deg_sh.at[idxr.at[k]], sem2).wait()
            return carry
        lax.fori_loop(0, HC, _drain_deg, 0)

    plsc.subcore_barrier()

    pltpu.sync_copy(accum_sh.at[pl.ds(r0, ROWS_PER_TILE)],
                    sum_out.at[c, pl.ds(r0, ROWS_PER_TILE)])
    pltpu.sync_copy(deg_sh.at[pl.ds(r0, ROWS_PER_TILE)], degv)
    pltpu.sync_copy(degv, deg_out.at[pl.ds(c * NP + r0, ROWS_PER_TILE)])


ROW_BLK = 2000
N_BLKS = N_NODES // ROW_BLK


def _tc_project(sp_ref, dp_ref, wt_ref, b_ref, o_ref):
    s = sp_ref[0] + sp_ref[1]
    d = dp_ref[0] + dp_ref[1]
    agg = s / jnp.maximum(d, 1.0)
    o_ref[...] = (
        jnp.dot(agg, wt_ref[...], preferred_element_type=jnp.float32)
        + b_ref[...]
    )


def kernel(x, edge_index, W, b):
    pad = E_PAD - N_EDGES
    pad_iota = jnp.arange(pad, dtype=jnp.int32)
    pads = jnp.stack([N_NODES + (pad_iota % (NP - N_NODES)),
                      pad_iota % N_NODES])
    edges = jnp.concatenate([edge_index.astype(jnp.int32), pads], axis=1)
    edges = edges.reshape(2, NW, CPW, CHUNK)

    sum_p, deg_p = _sc_aggregate(x, edges)

    out = pl.pallas_call(
        _tc_project,
        grid=(N_BLKS,),
        in_specs=[
            pl.BlockSpec((NC, ROW_BLK, FEATS), lambda i: (0, i, 0)),
            pl.BlockSpec((NC, ROW_BLK, 1), lambda i: (0, i, 0)),
            pl.BlockSpec((FEATS, FEATS), lambda i: (0, 0)),
            pl.BlockSpec((1, FEATS), lambda i: (0, 0)),
        ],
        out_specs=pl.BlockSpec((ROW_BLK, FEATS), lambda i: (i, 0)),
        out_shape=jax.ShapeDtypeStruct((N_NODES, FEATS), jnp.float32),
    )(sum_p, deg_p.reshape(NC, NP, 1), W.T, b.reshape(1, FEATS))
    return out


# speedup vs baseline: 14.5421x; 1.1393x over previous
import functools

import jax
import jax.numpy as jnp
from jax import lax
from jax.experimental import pallas as pl
from jax.experimental.pallas import tpu as pltpu
from jax.experimental.pallas import tpu_sc as plsc

N_NODES = 10000
N_EDGES = 320000
FEATS = 128

NC = 2
NS = 16
NW = NC * NS

CHUNK = 64
CPW = 160
EDGES_PER_W = CPW * CHUNK
E_PAD = EDGES_PER_W * NW
NBUF = 4

NP = 10240
ROWS_PER_TILE = NP // NS

_mesh = plsc.VectorSubcoreMesh(core_axis_name="c", subcore_axis_name="s")


@functools.partial(
    pl.kernel,
    out_type=[
        jax.ShapeDtypeStruct((NC, NP, FEATS), jnp.float32),
        jax.ShapeDtypeStruct((NC * NP,), jnp.float32),
    ],
    mesh=_mesh,
    scratch_types=[
        pltpu.VMEM((CPW // 4, CHUNK), jnp.int32),
        pltpu.VMEM((CPW // 4, CHUNK), jnp.int32),
        pltpu.VMEM((CHUNK, FEATS), jnp.float32),
        pltpu.VMEM((CHUNK, FEATS), jnp.float32),
        pltpu.VMEM((CHUNK, FEATS), jnp.float32),
        pltpu.VMEM((CHUNK, FEATS), jnp.float32),
        pltpu.VMEM((CHUNK,), jnp.float32),
        pltpu.VMEM((ROWS_PER_TILE,), jnp.float32),
        pltpu.SemaphoreType.DMA,
        pltpu.SemaphoreType.DMA,
        pltpu.SemaphoreType.DMA,
        pltpu.SemaphoreType.DMA,
        pltpu.SemaphoreType.DMA,
        pltpu.VMEM_SHARED((NP, FEATS), jnp.float32),
        pltpu.VMEM_SHARED((NP,), jnp.float32),
    ],
)
def _sc_aggregate(x_hbm, edges_hbm,
                  sum_out, deg_out,
                  idxc, idxr, buf0, buf1, buf2, buf3, onesv, degv,
                  sem0, sem1, sem2, sem3, semd,
                  accum_sh, deg_sh):
    bufs = (buf0, buf1, buf2, buf3)
    sems = (sem0, sem1, sem2, sem3)
    c = lax.axis_index("c")
    s = lax.axis_index("s")
    wid = s * NC + c
    r0 = s * ROWS_PER_TILE

    def _zero_buf(i, carry):
        for j in range(8):
            buf0[i, pl.ds(j * 16, 16)] = jnp.zeros((16,), jnp.float32)
        return carry
    lax.fori_loop(0, CHUNK, _zero_buf, 0)
    for j in range(ROWS_PER_TILE // CHUNK):
        pltpu.sync_copy(buf0, accum_sh.at[pl.ds(r0 + j * CHUNK, CHUNK)])

    def _zero_deg(i, carry):
        degv[pl.ds(i * 16, 16)] = jnp.zeros((16,), jnp.float32)
        return carry
    lax.fori_loop(0, ROWS_PER_TILE // 16, _zero_deg, 0)
    pltpu.sync_copy(degv, deg_sh.at[pl.ds(r0, ROWS_PER_TILE)])

    def _fill_ones(t, carry):
        onesv[pl.ds(t * 16, 16)] = jnp.ones((16,), jnp.float32)
        return carry
    lax.fori_loop(0, CHUNK // 16, _fill_ones, 0)

    plsc.subcore_barrier()

    HC = CPW // 4
    for h in range(4):
        pltpu.sync_copy(edges_hbm.at[1, wid, pl.ds(h * HC, HC)], idxc)
        pltpu.sync_copy(edges_hbm.at[0, wid, pl.ds(h * HC, HC)], idxr)
        for j in range(NBUF - 1):
            pltpu.async_copy(x_hbm.at[idxc.at[j]], bufs[j], sems[j])

        def _edge_group(g, carry):
            for j in range(NBUF):
                k = g * NBUF + j
                kn = jnp.minimum(k + NBUF - 1, HC - 1)
                jn = (j + NBUF - 1) % NBUF
                pltpu.make_async_copy(x_hbm.at[idxc.at[k]],
                                      bufs[j], sems[j]).wait()
                pltpu.async_copy(x_hbm.at[idxc.at[kn]], bufs[jn], sems[jn])
                pltpu.sync_copy(bufs[j], accum_sh.at[idxr.at[k]], add=True)
                pltpu.async_copy(onesv, deg_sh.at[idxr.at[k]], semd, add=True)
            return carry
        lax.fori_loop(0, HC // NBUF, _edge_group, 0)

        for j in range(NBUF - 1):
            pltpu.make_async_copy(x_hbm.at[idxc.at[HC - 1]],
                                  bufs[j], sems[j]).wait()

        def _drain_deg(k, carry):
            pltpu.make_async_copy(onesv, deg_sh.at[idxr.at[k]], semd).wait()
            return carry
        lax.fori_loop(0, HC, _drain_deg, 0)

    plsc.subcore_barrier()

    pltpu.sync_copy(accum_sh.at[pl.ds(r0, ROWS_PER_TILE)],
                    sum_out.at[c, pl.ds(r0, ROWS_PER_TILE)])
    pltpu.sync_copy(deg_sh.at[pl.ds(r0, ROWS_PER_TILE)], degv)
    pltpu.sync_copy(degv, deg_out.at[pl.ds(c * NP + r0, ROWS_PER_TILE)])


ROW_BLK = 2000
N_BLKS = N_NODES // ROW_BLK


def _tc_project(sp_ref, dp_ref, wt_ref, b_ref, o_ref):
    s = sp_ref[0] + sp_ref[1]
    d = dp_ref[0] + dp_ref[1]
    agg = s / jnp.maximum(d, 1.0)
    o_ref[...] = (
        jnp.dot(agg, wt_ref[...], preferred_element_type=jnp.float32)
        + b_ref[...]
    )


def kernel(x, edge_index, W, b):
    pad = E_PAD - N_EDGES
    pad_iota = jnp.arange(pad, dtype=jnp.int32)
    pads = jnp.stack([N_NODES + (pad_iota % (NP - N_NODES)),
                      pad_iota % N_NODES])
    edges = jnp.concatenate([edge_index.astype(jnp.int32), pads], axis=1)
    edges = edges.reshape(2, NW, CPW, CHUNK)

    sum_p, deg_p = _sc_aggregate(x, edges)

    out = pl.pallas_call(
        _tc_project,
        grid=(N_BLKS,),
        in_specs=[
            pl.BlockSpec((NC, ROW_BLK, FEATS), lambda i: (0, i, 0)),
            pl.BlockSpec((NC, ROW_BLK, 1), lambda i: (0, i, 0)),
            pl.BlockSpec((FEATS, FEATS), lambda i: (0, 0)),
            pl.BlockSpec((1, FEATS), lambda i: (0, 0)),
        ],
        out_specs=pl.BlockSpec((ROW_BLK, FEATS), lambda i: (i, 0)),
        out_shape=jax.ShapeDtypeStruct((N_NODES, FEATS), jnp.float32),
    )(sum_p, deg_p.reshape(NC, NP, 1), W.T, b.reshape(1, FEATS))
    return out


# speedup vs baseline: 14.8461x; 1.0209x over previous
import functools

import jax
import jax.numpy as jnp
from jax import lax
from jax.experimental import pallas as pl
from jax.experimental.pallas import tpu as pltpu
from jax.experimental.pallas import tpu_sc as plsc

N_NODES = 10000
N_EDGES = 320000
FEATS = 128

NC = 2
NS = 16
NW = NC * NS

CHUNK = 64
CPW = 160
EDGES_PER_W = CPW * CHUNK
E_PAD = EDGES_PER_W * NW
NBUF = 4

NP = 10240
ROWS_PER_TILE = NP // NS

_mesh = plsc.VectorSubcoreMesh(core_axis_name="c", subcore_axis_name="s")


@functools.partial(
    pl.kernel,
    out_type=[
        jax.ShapeDtypeStruct((NC, NP, FEATS), jnp.float32),
        jax.ShapeDtypeStruct((NC * NP,), jnp.float32),
    ],
    mesh=_mesh,
    scratch_types=[
        pltpu.VMEM((CPW // 4, CHUNK), jnp.int32),
        pltpu.VMEM((CPW // 4, CHUNK), jnp.int32),
        pltpu.VMEM((CHUNK, FEATS), jnp.float32),
        pltpu.VMEM((CHUNK, FEATS), jnp.float32),
        pltpu.VMEM((CHUNK, FEATS), jnp.float32),
        pltpu.VMEM((CHUNK, FEATS), jnp.float32),
        pltpu.VMEM((CHUNK,), jnp.float32),
        pltpu.VMEM((ROWS_PER_TILE,), jnp.float32),
        pltpu.SemaphoreType.DMA,
        pltpu.SemaphoreType.DMA,
        pltpu.SemaphoreType.DMA,
        pltpu.SemaphoreType.DMA,
        pltpu.SemaphoreType.DMA,
        pltpu.VMEM_SHARED((NP, FEATS), jnp.float32),
        pltpu.VMEM_SHARED((NP,), jnp.float32),
    ],
)
def _sc_aggregate(x_hbm, edges_hbm,
                  sum_out, deg_out,
                  idxc, idxr, buf0, buf1, buf2, buf3, onesv, degv,
                  sem0, sem1, sem2, sem3, semd,
                  accum_sh, deg_sh):
    bufs = (buf0, buf1, buf2, buf3)
    sems = (sem0, sem1, sem2, sem3)
    c = lax.axis_index("c")
    s = lax.axis_index("s")
    wid = s * NC + c
    r0 = s * ROWS_PER_TILE

    def _zero_buf(i, carry):
        for j in range(8):
            buf0[i, pl.ds(j * 16, 16)] = jnp.zeros((16,), jnp.float32)
        return carry
    lax.fori_loop(0, CHUNK, _zero_buf, 0)
    for j in range(ROWS_PER_TILE // CHUNK):
        pltpu.sync_copy(buf0, accum_sh.at[pl.ds(r0 + j * CHUNK, CHUNK)])

    def _zero_deg(i, carry):
        degv[pl.ds(i * 16, 16)] = jnp.zeros((16,), jnp.float32)
        return carry
    lax.fori_loop(0, ROWS_PER_TILE // 16, _zero_deg, 0)
    pltpu.sync_copy(degv, deg_sh.at[pl.ds(r0, ROWS_PER_TILE)])

    def _fill_ones(t, carry):
        onesv[pl.ds(t * 16, 16)] = jnp.ones((16,), jnp.float32)
        return carry
    lax.fori_loop(0, CHUNK // 16, _fill_ones, 0)

    plsc.subcore_barrier()

    HC = CPW // 4
    for h in range(4):
        pltpu.sync_copy(edges_hbm.at[1, wid, pl.ds(h * HC, HC)], idxc)
        pltpu.sync_copy(edges_hbm.at[0, wid, pl.ds(h * HC, HC)], idxr)
        for j in range(NBUF - 1):
            pltpu.async_copy(x_hbm.at[idxc.at[j]], bufs[j], sems[j])

        def _edge_group(g, carry):
            for j in range(NBUF):
                k = g * NBUF + j
                kn = k + NBUF - 1
                jn = (j + NBUF - 1) % NBUF
                pltpu.make_async_copy(x_hbm.at[idxc.at[k]],
                                      bufs[j], sems[j]).wait()

                @pl.when(kn < HC)
                def _issue():
                    pltpu.async_copy(x_hbm.at[idxc.at[kn]],
                                     bufs[jn], sems[jn])
                pltpu.sync_copy(bufs[j], accum_sh.at[idxr.at[k]], add=True)
                pltpu.async_copy(onesv, deg_sh.at[idxr.at[k]], semd, add=True)
            return carry
        lax.fori_loop(0, HC // NBUF, _edge_group, 0)

        def _drain_deg(k, carry):
            pltpu.make_async_copy(onesv, deg_sh.at[idxr.at[k]], semd).wait()
            return carry
        lax.fori_loop(0, HC, _drain_deg, 0)

    plsc.subcore_barrier()

    pltpu.sync_copy(accum_sh.at[pl.ds(r0, ROWS_PER_TILE)],
                    sum_out.at[c, pl.ds(r0, ROWS_PER_TILE)])
    pltpu.sync_copy(deg_sh.at[pl.ds(r0, ROWS_PER_TILE)], degv)
    pltpu.sync_copy(degv, deg_out.at[pl.ds(c * NP + r0, ROWS_PER_TILE)])


ROW_BLK = 2000
N_BLKS = N_NODES // ROW_BLK


def _tc_project(sp_ref, dp_ref, wt_ref, b_ref, o_ref):
    s = sp_ref[0] + sp_ref[1]
    d = dp_ref[0] + dp_ref[1]
    agg = s / jnp.maximum(d, 1.0)
    o_ref[...] = (
        jnp.dot(agg, wt_ref[...], preferred_element_type=jnp.float32)
        + b_ref[...]
    )


def kernel(x, edge_index, W, b):
    pad = E_PAD - N_EDGES
    pad_iota = jnp.arange(pad, dtype=jnp.int32)
    pads = jnp.stack([N_NODES + (pad_iota % (NP - N_NODES)),
                      pad_iota % N_NODES])
    edges = jnp.concatenate([edge_index.astype(jnp.int32), pads], axis=1)
    edges = edges.reshape(2, NW, CPW, CHUNK)

    sum_p, deg_p = _sc_aggregate(x, edges)

    out = pl.pallas_call(
        _tc_project,
        grid=(N_BLKS,),
        in_specs=[
            pl.BlockSpec((NC, ROW_BLK, FEATS), lambda i: (0, i, 0)),
            pl.BlockSpec((NC, ROW_BLK, 1), lambda i: (0, i, 0)),
            pl.BlockSpec((FEATS, FEATS), lambda i: (0, 0)),
            pl.BlockSpec((1, FEATS), lambda i: (0, 0)),
        ],
        out_specs=pl.BlockSpec((ROW_BLK, FEATS), lambda i: (i, 0)),
        out_shape=jax.ShapeDtypeStruct((N_NODES, FEATS), jnp.float32),
    )(sum_p, deg_p.reshape(NC, NP, 1), W.T, b.reshape(1, FEATS))
    return out
